# baseline (speedup 1.0000x reference)
import jax
import jax.numpy as jnp
from jax.experimental import pallas as pl


_BLK = 4096


def _fp_kernel(xyz1_ref, xyz2t_ref, points1_ref, points2_ref,
               w0a_ref, w0b_ref, b0_ref, w1_ref, b1_ref, out_ref):
    x1 = xyz1_ref[0]
    x2 = xyz2t_ref[0]
    dx = x1[:, 0:1] - x2[0:1, :]
    dy = x1[:, 1:2] - x2[1:2, :]
    dz = x1[:, 2:3] - x2[2:3, :]
    d = dx * dx + dy * dy + dz * dz

    inf = jnp.float32(jnp.inf)
    m1 = jnp.min(d, axis=1, keepdims=True)
    m2 = jnp.min(jnp.where(d > m1, d, inf), axis=1, keepdims=True)
    m3 = jnp.min(jnp.where(d > m2, d, inf), axis=1, keepdims=True)

    inv_sum = (1.0 / jnp.maximum(m1, 1e-10) + 1.0 / jnp.maximum(m2, 1e-10)
               + 1.0 / jnp.maximum(m3, 1e-10))
    sel = jnp.where(d <= m3, 1.0 / jnp.maximum(d, 1e-10), 0.0)
    interp = jnp.dot(sel, points2_ref[0],
                     preferred_element_type=jnp.float32) * (1.0 / inv_sum)

    h = jnp.maximum(
        jnp.dot(interp, w0a_ref[...], preferred_element_type=jnp.float32)
        + jnp.dot(points1_ref[0], w0b_ref[...], preferred_element_type=jnp.float32)
        + b0_ref[...], 0.0)
    out_ref[0] = jnp.maximum(
        jnp.dot(h, w1_ref[...], preferred_element_type=jnp.float32)
        + b1_ref[...], 0.0)


@jax.jit
def kernel(xyz1, points1, xyz2, points2, W0, b0, W1, b1):
    B, N1, _ = xyz1.shape
    _, N2, C2 = points2.shape
    C1 = points1.shape[2]
    xyz2t = jnp.swapaxes(xyz2, 1, 2)
    w0a, w0b = W0[:C2], W0[C2:]
    b0r = b0.reshape(1, -1)
    b1r = b1.reshape(1, -1)
    grid = (B, N1 // _BLK)
    return pl.pallas_call(
        _fp_kernel,
        grid=grid,
        in_specs=[
            pl.BlockSpec((1, _BLK, 3), lambda b, j: (b, j, 0)),
            pl.BlockSpec((1, 3, N2), lambda b, j: (b, 0, 0)),
            pl.BlockSpec((1, _BLK, C1), lambda b, j: (b, j, 0)),
            pl.BlockSpec((1, N2, C2), lambda b, j: (b, 0, 0)),
            pl.BlockSpec((C2, W0.shape[1]), lambda b, j: (0, 0)),
            pl.BlockSpec((C1, W0.shape[1]), lambda b, j: (0, 0)),
            pl.BlockSpec((1, W0.shape[1]), lambda b, j: (0, 0)),
            pl.BlockSpec(W1.shape, lambda b, j: (0, 0)),
            pl.BlockSpec((1, W1.shape[1]), lambda b, j: (0, 0)),
        ],
        out_specs=pl.BlockSpec((1, _BLK, W1.shape[1]), lambda b, j: (b, j, 0)),
        out_shape=jax.ShapeDtypeStruct((B, N1, W1.shape[1]), jnp.float32),
    )(xyz1, xyz2t, points1, points2, w0a, w0b, b0r, W1, b1r)
